# baseline (speedup 1.0000x reference)
import functools

import jax
import jax.numpy as jnp
from jax import lax
from jax.experimental import pallas as pl
from jax.experimental.pallas import tpu as pltpu
from jax.experimental.pallas import tpu_sc as plsc

N = 10000
E = 320000
F_IN = 128
HID = 16
C = 40
CP = 48

NC = 2
NSUB = 16
NW = NC * NSUB
EW = E // NW
SB = 125
NSTREAM = EW // SB
ROWS_PER_SUB = N // NSUB


def _worker_id():
    cid = lax.axis_index("c")
    sid = lax.axis_index("s")
    return cid, sid


def _deg_body(dst_hbm, ones_hbm, zeros_hbm, out_hbm, dst_v, ones_v, acc_sh, dsem):
    cid, sid = _worker_id()
    wid = sid * NC + cid

    @pl.when(sid == 0)
    def _():
        pltpu.sync_copy(zeros_hbm, acc_sh)

    pltpu.sync_copy(ones_hbm, ones_v)
    pltpu.sync_copy(dst_hbm.at[wid], dst_v)
    plsc.subcore_barrier()

    def step(j, _):
        pltpu.async_copy(ones_v, acc_sh.at[dst_v.at[j]], dsem, add=True)
        return ()

    lax.fori_loop(0, NSTREAM, step, ())

    def drain(j, _):
        pltpu.make_async_copy(ones_v, acc_sh.at[dst_v.at[j]], dsem).wait()
        return ()

    lax.fori_loop(0, NSTREAM, drain, ())
    plsc.subcore_barrier()

    @pl.when(sid == 0)
    def _():
        pltpu.sync_copy(acc_sh, out_hbm.at[cid])


_deg_call = pl.kernel(
    _deg_body,
    out_type=jax.ShapeDtypeStruct((NC, N), jnp.float32),
    mesh=plsc.VectorSubcoreMesh(core_axis_name="c", subcore_axis_name="s"),
    scratch_types=[
        pltpu.VMEM((NSTREAM, SB), jnp.int32),
        pltpu.VMEM((SB,), jnp.float32),
        pltpu.VMEM_SHARED((N,), jnp.float32),
        pltpu.SemaphoreType.DMA,
    ],
    compiler_params=pltpu.CompilerParams(use_tc_tiling_on_sc=False),
)


def _spmm_body(width, nslot, ngin, src_hbm, dst_hbm, table_hbm, zeros_hbm,
               out_hbm, src_v, dst_v, rows_v, acc_sh, ssem, *gsems):
    cid, sid = _worker_id()
    wid = sid * NC + cid
    rps = ROWS_PER_SUB
    pltpu.sync_copy(zeros_hbm, acc_sh.at[pl.ds(sid * rps, rps)])
    pltpu.sync_copy(src_hbm.at[wid], src_v)
    pltpu.sync_copy(dst_hbm.at[wid], dst_v)
    plsc.subcore_barrier()

    def start_g(j, slot):
        pltpu.async_copy(table_hbm.at[src_v.at[j]], rows_v.at[slot], gsems[slot])

    def wait_g(j, slot):
        pltpu.make_async_copy(table_hbm.at[src_v.at[j]], rows_v.at[slot],
                              gsems[slot]).wait()

    def start_s(j, slot):
        pltpu.async_copy(rows_v.at[slot], acc_sh.at[dst_v.at[j]], ssem, add=True)

    def wait_s_one(j, slot):
        pltpu.make_async_copy(rows_v.at[slot], acc_sh.at[dst_v.at[j]], ssem).wait()

    for s in range(ngin):
        start_g(s, s)

    def step(i, _):
        for s in range(nslot):
            j = i * nslot + s
            wait_g(j, s)
            start_s(j, s)

            @pl.when(j >= ngin)
            def _():
                wait_s_one(j, s)

            @pl.when(j + ngin < NSTREAM)
            def _():
                start_g(j + ngin, (s + ngin) % nslot)

        return ()

    lax.fori_loop(0, NSTREAM // nslot, step, ())
    for s in range(ngin):
        wait_s_one(0, s)
    plsc.subcore_barrier()
    pltpu.sync_copy(acc_sh.at[pl.ds(sid * rps, rps)], out_hbm.at[cid, sid])


def _make_spmm(width, nslot, ngin):
    return pl.kernel(
        functools.partial(_spmm_body, width, nslot, ngin),
        out_type=jax.ShapeDtypeStruct((NC, NSUB, ROWS_PER_SUB, width), jnp.float32),
        mesh=plsc.VectorSubcoreMesh(core_axis_name="c", subcore_axis_name="s"),
        scratch_types=[
            pltpu.VMEM((NSTREAM, SB), jnp.int32),
            pltpu.VMEM((NSTREAM, SB), jnp.int32),
            pltpu.VMEM((nslot, SB, width), jnp.float32),
            pltpu.VMEM_SHARED((N, width), jnp.float32),
            pltpu.SemaphoreType.DMA,
        ] + [pltpu.SemaphoreType.DMA] * nslot,
        compiler_params=pltpu.CompilerParams(use_tc_tiling_on_sc=False),
    )


_spmm16 = _make_spmm(HID, 20, 10)
_spmm40 = _make_spmm(C, 16, 8)


def _tc1_body(x_ref, w1_ref, degp_ref, h1n_ref, dinv_ref):
    deg = degp_ref[0, :] + degp_ref[1, :] + 1.0
    dinv = lax.rsqrt(deg)[:, None]
    h1 = jnp.dot(x_ref[...], w1_ref[...], preferred_element_type=jnp.float32)
    dinv_ref[...] = dinv
    h1n_ref[...] = h1 * dinv


_tc1 = pl.pallas_call(
    _tc1_body,
    out_shape=(jax.ShapeDtypeStruct((N, HID), jnp.float32),
               jax.ShapeDtypeStruct((N, 1), jnp.float32)),
)


def _tc2_body(agg_ref, h1n_ref, dinv_ref, b1_ref, w2_ref, h2n_ref):
    dinv = dinv_ref[...]
    z1 = dinv * (agg_ref[0] + agg_ref[1] + h1n_ref[...]) + b1_ref[...]
    z1 = jnp.maximum(z1, 0.0)
    h2 = jnp.dot(z1, w2_ref[...], preferred_element_type=jnp.float32)
    h2n_ref[...] = h2 * dinv


_tc2 = pl.pallas_call(
    _tc2_body,
    out_shape=jax.ShapeDtypeStruct((N, C), jnp.float32),
)


def _tc3_body(agg_ref, h2n_ref, dinv_ref, b2_ref, out_ref):
    z = dinv_ref[...] * (agg_ref[0] + agg_ref[1] + h2n_ref[...]) + b2_ref[...]
    m = jnp.max(z, axis=1, keepdims=True)
    e = jnp.exp(z - m)
    lse = jnp.log(jnp.sum(e, axis=1, keepdims=True))
    out_ref[...] = z - m - lse


_tc3 = pl.pallas_call(
    _tc3_body,
    out_shape=jax.ShapeDtypeStruct((N, C), jnp.float32),
)


def kernel(x, edge_index, W1, b1, W2, b2):
    src = edge_index[0].reshape(NW, NSTREAM, SB)
    dst = edge_index[1].reshape(NW, NSTREAM, SB)

    ones1 = jnp.ones((SB,), jnp.float32)
    zerosN = jnp.zeros((N,), jnp.float32)
    zeros16 = jnp.zeros((ROWS_PER_SUB, HID), jnp.float32)
    zeros40 = jnp.zeros((ROWS_PER_SUB, C), jnp.float32)

    degp = _deg_call(dst, ones1, zerosN)
    h1n, dinv = _tc1(x, W1, degp)
    agg1 = _spmm16(src, dst, h1n, zeros16).reshape(NC, N, HID)
    h2n = _tc2(agg1, h1n, dinv, b1.reshape(1, HID), W2)
    agg2 = _spmm40(src, dst, h2n, zeros40).reshape(NC, N, C)
    out = _tc3(agg2, h2n, dinv, b2.reshape(1, C))
    return out
